# baseline (speedup 1.0000x reference)
import functools

import jax
import jax.numpy as jnp
from jax import lax
from jax.experimental import pallas as pl
from jax.experimental.pallas import tpu as pltpu
from jax.experimental.pallas import tpu_sc as plsc

T = 2048
D = 768
FFN = 3072
E = 8
BLK = 128
NBLK = 40
NPAD = NBLK * BLK
DW = D // 2

_INTERPRET = False

_NC = 2
_NS = 16
_NW = _NC * _NS
_CHUNK = T // _NW


def _sc_mesh():
    return plsc.VectorSubcoreMesh(core_axis_name="c", subcore_axis_name="s")


def _dispatch_body(sel_ref, pos0_ref, pos1_ref, be_ref):
    lane = lax.broadcasted_iota(jnp.int32, (T, E), 1)

    def _bc(v):
        return jnp.broadcast_to(v, (T, E))

    sel0 = sel_ref[:, 0:1]
    sel1 = sel_ref[:, 1:2]
    oh0 = (lane == _bc(sel0)).astype(jnp.bfloat16)
    oh1 = (lane == _bc(sel1)).astype(jnp.bfloat16)
    ri = lax.broadcasted_iota(jnp.int32, (T, T), 0)
    ci = lax.broadcasted_iota(jnp.int32, (T, T), 1)
    lt = (ci < ri).astype(jnp.bfloat16)
    rank0 = lax.dot_general(lt, oh0, (((1,), (0,)), ((), ())),
                            preferred_element_type=jnp.float32)
    rank1 = lax.dot_general(lt, oh1, (((1,), (0,)), ((), ())),
                            preferred_element_type=jnp.float32)
    tot0 = jnp.sum(oh0.astype(jnp.float32), axis=0, keepdims=True)
    tot1 = jnp.sum(oh1.astype(jnp.float32), axis=0, keepdims=True)
    rank1 = rank1 + _bc(tot0)
    cnt = tot0 + tot1
    nb = jnp.floor((cnt + (BLK - 1)) * (1.0 / BLK))
    lane8r = lax.broadcasted_iota(jnp.int32, (E, E), 0)
    lane8c = lax.broadcasted_iota(jnp.int32, (E, E), 1)
    lt8 = (lane8c < lane8r).astype(jnp.float32)
    bs = lax.dot_general(nb, lt8, (((1,), (1,)), ((), ())),
                         preferred_element_type=jnp.float32)
    off = bs * float(BLK)
    pos0 = jnp.sum((_bc(off) + rank0) * oh0.astype(jnp.float32),
                   axis=1, keepdims=True)
    pos1 = jnp.sum((_bc(off) + rank1) * oh1.astype(jnp.float32),
                   axis=1, keepdims=True)
    pos0_ref[...] = pos0.astype(jnp.int32)
    pos1_ref[...] = pos1.astype(jnp.int32)
    biota = lax.broadcasted_iota(jnp.int32, (128, E), 0)
    bsi = bs.astype(jnp.int32)
    ge = (biota >= jnp.broadcast_to(bsi, (128, E))).astype(jnp.float32)
    be = jnp.sum(ge, axis=1, keepdims=True) - 1.0
    be_ref[...] = be.astype(jnp.int32)


def _dispatch(sel):
    return pl.pallas_call(
        _dispatch_body,
        out_shape=(
            jax.ShapeDtypeStruct((T, 1), jnp.int32),
            jax.ShapeDtypeStruct((T, 1), jnp.int32),
            jax.ShapeDtypeStruct((128, 1), jnp.int32),
        ),
        interpret=_INTERPRET,
    )(sel)


@functools.lru_cache(maxsize=None)
def _make_sc_scatter():
    @functools.partial(
        pl.kernel,
        mesh=_sc_mesh(),
        out_type=jax.ShapeDtypeStruct((NPAD, DW), jnp.int32),
        scratch_types=[
            pltpu.VMEM((_CHUNK,), jnp.int32),
            pltpu.VMEM((_CHUNK,), jnp.int32),
            pltpu.VMEM((_CHUNK, DW), jnp.int32),
            pltpu.SemaphoreType.DMA,
        ],
    )
    def _sc_scatter(hs_hbm, pos0_hbm, pos1_hbm, out_hbm,
                    i0_v, i1_v, rows_v, sem):
        wid = lax.axis_index("s") * _NC + lax.axis_index("c")
        base = wid * _CHUNK
        pltpu.sync_copy(hs_hbm.at[pl.ds(base, _CHUNK)], rows_v)
        pltpu.sync_copy(pos0_hbm.at[pl.ds(base, _CHUNK)], i0_v)
        pltpu.sync_copy(pos1_hbm.at[pl.ds(base, _CHUNK)], i1_v)
        pltpu.async_copy(rows_v, out_hbm.at[i0_v], sem).wait()
        pltpu.async_copy(rows_v, out_hbm.at[i1_v], sem).wait()

    return _sc_scatter


def _gffn_body(be_ref, x_ref, wg_ref, wu_ref, wd_ref, y_ref):
    x = x_ref[...]
    a = lax.dot_general(x, wg_ref[0], (((1,), (1,)), ((), ())),
                        preferred_element_type=jnp.float32).astype(jnp.bfloat16)
    b = lax.dot_general(x, wu_ref[0], (((1,), (1,)), ((), ())),
                        preferred_element_type=jnp.float32).astype(jnp.bfloat16)
    h = (a * jax.nn.sigmoid(a)) * b
    y_ref[...] = lax.dot_general(h, wd_ref[0], (((1,), (1,)), ((), ())),
                                 preferred_element_type=jnp.float32
                                 ).astype(jnp.bfloat16)


def _grouped_ffn(be, x_sorted, w_gate, w_up, w_down):
    grid_spec = pltpu.PrefetchScalarGridSpec(
        num_scalar_prefetch=1,
        grid=(NBLK,),
        in_specs=[
            pl.BlockSpec((BLK, D), lambda b, be: (b, 0)),
            pl.BlockSpec((1, FFN, D), lambda b, be: (be[b], 0, 0)),
            pl.BlockSpec((1, FFN, D), lambda b, be: (be[b], 0, 0)),
            pl.BlockSpec((1, D, FFN), lambda b, be: (be[b], 0, 0)),
        ],
        out_specs=pl.BlockSpec((BLK, D), lambda b, be: (b, 0)),
    )
    return pl.pallas_call(
        _gffn_body,
        grid_spec=grid_spec,
        out_shape=jax.ShapeDtypeStruct((NPAD, D), jnp.bfloat16),
        interpret=_INTERPRET,
    )(be, x_sorted, w_gate, w_up, w_down)


@functools.lru_cache(maxsize=None)
def _make_sc_combine():
    @functools.partial(
        pl.kernel,
        mesh=_sc_mesh(),
        out_type=jax.ShapeDtypeStruct((T, DW), jnp.int32),
        scratch_types=[
            pltpu.VMEM((_CHUNK,), jnp.int32),
            pltpu.VMEM((_CHUNK,), jnp.int32),
            pltpu.VMEM((_CHUNK, 16), jnp.float32),
            pltpu.VMEM((_CHUNK, 16), jnp.float32),
            pltpu.VMEM((_CHUNK, DW), jnp.int32),
            pltpu.VMEM((_CHUNK, DW), jnp.int32),
            pltpu.VMEM((_CHUNK, DW), jnp.int32),
            pltpu.SemaphoreType.DMA,
        ],
    )
    def _sc_combine(y_hbm, pos0_hbm, pos1_hbm, w0_hbm, w1_hbm, out_hbm,
                    i0_v, i1_v, w0_v, w1_v, r0_v, r1_v, o_v, sem):
        wid = lax.axis_index("s") * _NC + lax.axis_index("c")
        base = wid * _CHUNK
        pltpu.sync_copy(pos0_hbm.at[pl.ds(base, _CHUNK)], i0_v)
        pltpu.sync_copy(pos1_hbm.at[pl.ds(base, _CHUNK)], i1_v)
        pltpu.sync_copy(w0_hbm.at[pl.ds(base, _CHUNK)], w0_v)
        pltpu.sync_copy(w1_hbm.at[pl.ds(base, _CHUNK)], w1_v)
        pltpu.async_copy(y_hbm.at[i0_v], r0_v, sem).wait()
        pltpu.async_copy(y_hbm.at[i1_v], r1_v, sem).wait()

        topmask = jnp.int32(-65536)

        def _lo_f32(word):
            return lax.bitcast_convert_type(lax.shift_left(word, 16), jnp.float32)

        def _hi_f32(word):
            return lax.bitcast_convert_type(word & topmask, jnp.float32)

        def _rne_bits(x):
            u = lax.bitcast_convert_type(x, jnp.int32)
            r = u + jnp.int32(0x7FFF) + (lax.shift_right_logical(u, 16)
                                         & jnp.int32(1))
            return r & topmask

        def row(i, carry):
            w0b = w0_v[i, :]
            w1b = w1_v[i, :]
            for j in range(DW // 16):
                y0 = r0_v[i, pl.ds(j * 16, 16)]
                y1 = r1_v[i, pl.ds(j * 16, 16)]
                s_lo = w0b * _lo_f32(y0) + w1b * _lo_f32(y1)
                s_hi = w0b * _hi_f32(y0) + w1b * _hi_f32(y1)
                word = (lax.shift_right_logical(_rne_bits(s_lo), 16)
                        | _rne_bits(s_hi))
                o_v[i, pl.ds(j * 16, 16)] = word
            return carry

        lax.fori_loop(0, _CHUNK, row, 0)
        pltpu.sync_copy(o_v, out_hbm.at[pl.ds(base, _CHUNK)])

    return _sc_combine


@functools.partial(jax.jit, static_argnames=())
def kernel(hidden_states, gate_w, w_gate, w_up, w_down):
    bsz, seq, d = hidden_states.shape
    hs = hidden_states.reshape(-1, d)
    logits = (hs @ gate_w.T).astype(jnp.bfloat16)
    p = jax.nn.softmax(logits, axis=1)
    rw_topk, sel = jax.lax.top_k(p, 2)
    rw32 = rw_topk.astype(jnp.float32)
    rw32 = rw32 / rw32.sum(axis=-1, keepdims=True)
    w = rw32.astype(jnp.bfloat16)

    pos0, pos1, be = _dispatch(sel)
    pos0 = pos0.reshape(T)
    pos1 = pos1.reshape(T)
    be = be.reshape(128)[:NBLK]

    hs_i32 = lax.bitcast_convert_type(hs.reshape(T, DW, 2), jnp.int32)
    x_sorted_i32 = _make_sc_scatter()(hs_i32, pos0, pos1)
    x_sorted = lax.bitcast_convert_type(
        x_sorted_i32, jnp.bfloat16).reshape(NPAD, D)

    y = _grouped_ffn(be, x_sorted, w_gate, w_up, w_down)

    y_i32 = lax.bitcast_convert_type(y.reshape(NPAD, DW, 2), jnp.int32)
    w0rep = jnp.broadcast_to(w[:, 0:1].astype(jnp.float32), (T, 16))
    w1rep = jnp.broadcast_to(w[:, 1:2].astype(jnp.float32), (T, 16))
    out_i32 = _make_sc_combine()(y_i32, pos0, pos1, w0rep, w1rep)
    out = lax.bitcast_convert_type(out_i32, jnp.bfloat16).reshape(T, D)
    return out.reshape(bsz, seq, d), logits


# speedup vs baseline: 1.9870x; 1.9870x over previous
import functools

import jax
import jax.numpy as jnp
from jax import lax
from jax.experimental import pallas as pl
from jax.experimental.pallas import tpu as pltpu

T = 2048
D = 768
FFN = 3072
E = 8

_INTERPRET = False


def _logits_body(hs_ref, gw_ref, logits_ref):
    logits_ref[...] = lax.dot_general(
        hs_ref[...], gw_ref[...], (((1,), (1,)), ((), ())),
        preferred_element_type=jnp.float32).astype(jnp.bfloat16)


def _logits(hs, gate_w):
    return pl.pallas_call(
        _logits_body,
        out_shape=jax.ShapeDtypeStruct((T, E), jnp.bfloat16),
        interpret=_INTERPRET,
    )(hs, gate_w)


def _dispatch_body(sel_ref, w_ref, combine_ref):
    lane = lax.broadcasted_iota(jnp.int32, (T, E), 1)

    def _bc(v):
        return jnp.broadcast_to(v, (T, E))

    sel0 = sel_ref[:, 0:1]
    sel1 = sel_ref[:, 1:2]
    w0 = w_ref[:, 0:1].astype(jnp.float32)
    w1 = w_ref[:, 1:2].astype(jnp.float32)
    oh0 = (lane == _bc(sel0)).astype(jnp.float32)
    oh1 = (lane == _bc(sel1)).astype(jnp.float32)
    combine_ref[...] = (oh0 * _bc(w0) + oh1 * _bc(w1)).astype(jnp.bfloat16)


def _dispatch(sel, w):
    return pl.pallas_call(
        _dispatch_body,
        out_shape=jax.ShapeDtypeStruct((T, E), jnp.bfloat16),
        interpret=_INTERPRET,
    )(sel, w)


def _ffn_body(x_ref, wg_ref, wu_ref, wd_ref, comb_ref, out_ref):
    e = pl.program_id(1)
    x = x_ref[...]
    a = lax.dot_general(x, wg_ref[0], (((1,), (1,)), ((), ())),
                        preferred_element_type=jnp.float32).astype(jnp.bfloat16)
    b = lax.dot_general(x, wu_ref[0], (((1,), (1,)), ((), ())),
                        preferred_element_type=jnp.float32).astype(jnp.bfloat16)
    h = (a * jax.nn.sigmoid(a)) * b
    y = lax.dot_general(h, wd_ref[0], (((1,), (1,)), ((), ())),
                        preferred_element_type=jnp.float32).astype(jnp.bfloat16)
    lane = lax.broadcasted_iota(jnp.int32, comb_ref.shape, 1)
    c = jnp.sum(jnp.where(lane == e, comb_ref[...], jnp.bfloat16(0)),
                axis=1, keepdims=True)
    contrib = y * c

    @pl.when(e == 0)
    def _():
        out_ref[...] = contrib

    @pl.when(e > 0)
    def _():
        out_ref[...] = out_ref[...] + contrib


def _dense_moe(hs, w_gate, w_up, w_down, combine):
    tb = 2
    rows = T // tb
    return pl.pallas_call(
        _ffn_body,
        grid=(tb, E),
        in_specs=[
            pl.BlockSpec((rows, D), lambda t, e: (t, 0)),
            pl.BlockSpec((1, FFN, D), lambda t, e: (e, 0, 0)),
            pl.BlockSpec((1, FFN, D), lambda t, e: (e, 0, 0)),
            pl.BlockSpec((1, D, FFN), lambda t, e: (e, 0, 0)),
            pl.BlockSpec((rows, E), lambda t, e: (t, 0)),
        ],
        out_specs=pl.BlockSpec((rows, D), lambda t, e: (t, 0)),
        out_shape=jax.ShapeDtypeStruct((T, D), jnp.bfloat16),
        interpret=_INTERPRET,
    )(hs, w_gate, w_up, w_down, combine)


@functools.partial(jax.jit, static_argnames=())
def kernel(hidden_states, gate_w, w_gate, w_up, w_down):
    bsz, seq, d = hidden_states.shape
    hs = hidden_states.reshape(-1, d)
    logits = (hs @ gate_w.T).astype(jnp.bfloat16)
    p = jax.nn.softmax(logits, axis=1)
    rw_topk, sel = jax.lax.top_k(p, 2)
    rw32 = rw_topk.astype(jnp.float32)
    rw32 = rw32 / rw32.sum(axis=-1, keepdims=True)
    w = rw32.astype(jnp.bfloat16)
    combine = _dispatch(sel, w)
    out = _dense_moe(hs, w_gate, w_up, w_down, combine)
    return out.reshape(bsz, seq, d), logits


# speedup vs baseline: 1.9878x; 1.0004x over previous
import functools

import jax
import jax.numpy as jnp
from jax import lax
from jax.experimental import pallas as pl
from jax.experimental.pallas import tpu as pltpu

T = 2048
D = 768
FFN = 3072
E = 8

_INTERPRET = False


def _dispatch_body(sel_ref, w_ref, combine_ref):
    lane = lax.broadcasted_iota(jnp.int32, (T, E), 1)

    def _bc(v):
        return jnp.broadcast_to(v, (T, E))

    sel0 = sel_ref[:, 0:1]
    sel1 = sel_ref[:, 1:2]
    w0 = w_ref[:, 0:1].astype(jnp.float32)
    w1 = w_ref[:, 1:2].astype(jnp.float32)
    oh0 = (lane == _bc(sel0)).astype(jnp.float32)
    oh1 = (lane == _bc(sel1)).astype(jnp.float32)
    combine_ref[...] = (oh0 * _bc(w0) + oh1 * _bc(w1)).astype(jnp.bfloat16)


def _dispatch(sel, w):
    return pl.pallas_call(
        _dispatch_body,
        out_shape=jax.ShapeDtypeStruct((T, E), jnp.bfloat16),
        interpret=_INTERPRET,
    )(sel, w)


def _ffn_body(x_ref, wg_ref, wu_ref, wd_ref, comb_ref, out_ref):
    e = pl.program_id(1)
    x = x_ref[...]
    a = lax.dot_general(x, wg_ref[0], (((1,), (1,)), ((), ())),
                        preferred_element_type=jnp.float32).astype(jnp.bfloat16)
    b = lax.dot_general(x, wu_ref[0], (((1,), (1,)), ((), ())),
                        preferred_element_type=jnp.float32).astype(jnp.bfloat16)
    h = (a * jax.nn.sigmoid(a)) * b
    y = lax.dot_general(h, wd_ref[0], (((1,), (1,)), ((), ())),
                        preferred_element_type=jnp.float32).astype(jnp.bfloat16)
    lane = lax.broadcasted_iota(jnp.int32, comb_ref.shape, 1)
    c = jnp.sum(jnp.where(lane == e, comb_ref[...], jnp.bfloat16(0)),
                axis=1, keepdims=True)
    contrib = y * c

    @pl.when(e == 0)
    def _():
        out_ref[...] = contrib

    @pl.when(e > 0)
    def _():
        out_ref[...] = out_ref[...] + contrib


def _dense_moe(hs, w_gate, w_up, w_down, combine):
    tb = 2
    rows = T // tb
    return pl.pallas_call(
        _ffn_body,
        grid=(tb, E),
        in_specs=[
            pl.BlockSpec((rows, D), lambda t, e: (t, 0)),
            pl.BlockSpec((1, FFN, D), lambda t, e: (e, 0, 0)),
            pl.BlockSpec((1, FFN, D), lambda t, e: (e, 0, 0)),
            pl.BlockSpec((1, D, FFN), lambda t, e: (e, 0, 0)),
            pl.BlockSpec((rows, E), lambda t, e: (t, 0)),
        ],
        out_specs=pl.BlockSpec((rows, D), lambda t, e: (t, 0)),
        out_shape=jax.ShapeDtypeStruct((T, D), jnp.bfloat16),
        interpret=_INTERPRET,
    )(hs, w_gate, w_up, w_down, combine)


@functools.partial(jax.jit, static_argnames=())
def kernel(hidden_states, gate_w, w_gate, w_up, w_down):
    bsz, seq, d = hidden_states.shape
    hs = hidden_states.reshape(-1, d)
    logits = (hs @ gate_w.T).astype(jnp.bfloat16)
    p = jax.nn.softmax(logits, axis=1)
    rw_topk, sel = jax.lax.top_k(p, 2)
    rw32 = rw_topk.astype(jnp.float32)
    rw32 = rw32 / rw32.sum(axis=-1, keepdims=True)
    w = rw32.astype(jnp.bfloat16)
    combine = _dispatch(sel, w)
    out = _dense_moe(hs, w_gate, w_up, w_down, combine)
    return out.reshape(bsz, seq, d), logits
